# Initial kernel scaffold; baseline (speedup 1.0000x reference)
import jax
import jax.numpy as jnp
from jax.experimental import pallas as pl


def kernel(x, edge_index, W1, b1, g1, be1, W2, b2, g2, be2, Wf, bf):
    raise NotImplementedError("write your pallas kernel here")


# speedup vs baseline: 18.0543x; 18.0543x over previous
import functools

import jax
import jax.numpy as jnp
from jax import lax
from jax.experimental import pallas as pl
from jax.experimental.pallas import tpu as pltpu
from jax.experimental.pallas import tpu_sc as plsc

N = 10000
E = 320000
D = 128
H = 128

NC = 2
NS = 16
NW = NC * NS
EPW = E // NW
C = 80
NCHUNK = EPW // C
DEGW = 128

ZR = 624
TAIL = N - NS * ZR
TAIL_OFF = NS * ZR


def _striped_copy(src_ref, dst_ref, sid):
    pltpu.sync_copy(src_ref.at[pl.ds(sid * ZR, ZR)],
                    dst_ref.at[pl.ds(sid * ZR, ZR)])

    @pl.when(sid == 0)
    def _():
        pltpu.sync_copy(src_ref.at[pl.ds(TAIL_OFF, TAIL)],
                        dst_ref.at[pl.ds(TAIL_OFF, TAIL)])


@functools.lru_cache(maxsize=None)
def _sc_degree_kernel():
    mesh = plsc.VectorSubcoreMesh(core_axis_name="c", subcore_axis_name="s")
    return functools.partial(
        pl.kernel,
        mesh=mesh,
        out_type=jax.ShapeDtypeStruct((NC, N, DEGW), jnp.float32),
        scratch_types=[
            pltpu.VMEM((NCHUNK, C), jnp.int32),
            pltpu.VMEM((C, DEGW), jnp.float32),
            pltpu.VMEM_SHARED((N, DEGW), jnp.float32),
        ],
    )(_sc_degree_body)


def _sc_degree_body(dst_hbm, ones_hbm, zeros_hbm, out_hbm, dst_v, ones_v, acc_sh):
    cid = lax.axis_index("c")
    sid = lax.axis_index("s")
    wid = sid * NC + cid
    _striped_copy(zeros_hbm, acc_sh, sid)
    pltpu.sync_copy(dst_hbm.at[wid], dst_v)
    pltpu.sync_copy(ones_hbm, ones_v)
    plsc.subcore_barrier()

    @pl.loop(0, NCHUNK)
    def _(j):
        pltpu.sync_copy(ones_v, acc_sh.at[dst_v.at[j]], add=True)

    plsc.subcore_barrier()
    _striped_copy(acc_sh, out_hbm.at[cid], sid)


@functools.lru_cache(maxsize=None)
def _sc_scatter_kernel():
    mesh = plsc.VectorSubcoreMesh(core_axis_name="c", subcore_axis_name="s")
    return functools.partial(
        pl.kernel,
        mesh=mesh,
        out_type=jax.ShapeDtypeStruct((NC, N, H), jnp.float32),
        scratch_types=[
            pltpu.VMEM((NCHUNK, C), jnp.int32),
            pltpu.VMEM((NCHUNK, C), jnp.int32),
            pltpu.VMEM((C, H), jnp.float32),
            pltpu.SemaphoreType.DMA,
            pltpu.VMEM_SHARED((N, H), jnp.float32),
        ],
    )(_sc_scatter_body)


def _sc_scatter_body(y_hbm, src_hbm, dst_hbm, zeros_hbm, out_hbm,
                     src_v, dst_v, rows_v, sem, acc_sh):
    cid = lax.axis_index("c")
    sid = lax.axis_index("s")
    wid = sid * NC + cid
    _striped_copy(zeros_hbm, acc_sh, sid)
    pltpu.sync_copy(src_hbm.at[wid], src_v)
    pltpu.sync_copy(dst_hbm.at[wid], dst_v)
    plsc.subcore_barrier()

    @pl.loop(0, NCHUNK)
    def _(j):
        pltpu.async_copy(y_hbm.at[src_v.at[j]], rows_v, sem).wait()
        pltpu.sync_copy(rows_v, acc_sh.at[dst_v.at[j]], add=True)

    plsc.subcore_barrier()
    _striped_copy(acc_sh, out_hbm.at[cid], sid)


BN = 1000


def _mm_body(x_ref, w_ref, o_ref):
    o_ref[...] = jnp.dot(x_ref[...], w_ref[...],
                         preferred_element_type=jnp.float32)


def _tc_matmul(x, w):
    m, k = x.shape
    n = w.shape[1]
    return pl.pallas_call(
        _mm_body,
        grid=(m // BN,),
        in_specs=[pl.BlockSpec((BN, k), lambda i: (i, 0)),
                  pl.BlockSpec((k, n), lambda i: (0, 0))],
        out_specs=pl.BlockSpec((BN, n), lambda i: (i, 0)),
        out_shape=jax.ShapeDtypeStruct((m, n), jnp.float32),
    )(x, w)


def _scale_body(dp_ref, xw_ref, y_ref, dis_ref):
    deg = 1.0 + dp_ref[0, :, 0:1] + dp_ref[1, :, 0:1]
    dis = lax.rsqrt(deg)
    dis_ref[...] = dis
    y_ref[...] = xw_ref[...] * dis


def _tc_scale(degp, xw):
    return pl.pallas_call(
        _scale_body,
        grid=(N // BN,),
        in_specs=[pl.BlockSpec((NC, BN, DEGW), lambda i: (0, i, 0)),
                  pl.BlockSpec((BN, H), lambda i: (i, 0))],
        out_specs=[pl.BlockSpec((BN, H), lambda i: (i, 0)),
                   pl.BlockSpec((BN, 1), lambda i: (i, 0))],
        out_shape=[jax.ShapeDtypeStruct((N, H), jnp.float32),
                   jax.ShapeDtypeStruct((N, 1), jnp.float32)],
    )(degp, xw)


def _ln(h, g_ref, be_ref):
    m = jnp.mean(h, axis=-1, keepdims=True)
    c = h - m
    v = jnp.mean(c * c, axis=-1, keepdims=True)
    return c * lax.rsqrt(v + 1e-5) * g_ref[...] + be_ref[...]


def _mid_body(p_ref, y_ref, dis_ref, b_ref, g_ref, be_ref, w_ref,
              x1_ref, y2_ref):
    agg = p_ref[0] + p_ref[1] + y_ref[...]
    h = agg * dis_ref[...] + b_ref[...]
    x1 = jnp.maximum(_ln(h, g_ref, be_ref), 0.0)
    x1_ref[...] = x1
    y2_ref[...] = jnp.dot(x1, w_ref[...],
                          preferred_element_type=jnp.float32) * dis_ref[...]


def _tc_mid(p, y1, dis, b1, g1, be1, w2):
    return pl.pallas_call(
        _mid_body,
        grid=(N // BN,),
        in_specs=[pl.BlockSpec((NC, BN, H), lambda i: (0, i, 0)),
                  pl.BlockSpec((BN, H), lambda i: (i, 0)),
                  pl.BlockSpec((BN, 1), lambda i: (i, 0)),
                  pl.BlockSpec((1, H), lambda i: (0, 0)),
                  pl.BlockSpec((1, H), lambda i: (0, 0)),
                  pl.BlockSpec((1, H), lambda i: (0, 0)),
                  pl.BlockSpec((H, H), lambda i: (0, 0))],
        out_specs=[pl.BlockSpec((BN, H), lambda i: (i, 0)),
                   pl.BlockSpec((BN, H), lambda i: (i, 0))],
        out_shape=[jax.ShapeDtypeStruct((N, H), jnp.float32),
                   jax.ShapeDtypeStruct((N, H), jnp.float32)],
    )(p, y1, dis, b1, g1, be1, w2)


def _final_body(q_ref, y_ref, dis_ref, b_ref, g_ref, be_ref, x1_ref,
                wf_ref, bf_ref, o_ref):
    agg = q_ref[0] + q_ref[1] + y_ref[...]
    h = agg * dis_ref[...] + b_ref[...]
    x2 = jnp.maximum(_ln(h, g_ref, be_ref), 0.0) + x1_ref[...]
    o_ref[...] = jnp.dot(x2, wf_ref[...],
                         preferred_element_type=jnp.float32) + bf_ref[...]


def _tc_final(q, y2, dis, b2, g2, be2, x1, wf, bf):
    return pl.pallas_call(
        _final_body,
        grid=(N // BN,),
        in_specs=[pl.BlockSpec((NC, BN, H), lambda i: (0, i, 0)),
                  pl.BlockSpec((BN, H), lambda i: (i, 0)),
                  pl.BlockSpec((BN, 1), lambda i: (i, 0)),
                  pl.BlockSpec((1, H), lambda i: (0, 0)),
                  pl.BlockSpec((1, H), lambda i: (0, 0)),
                  pl.BlockSpec((1, H), lambda i: (0, 0)),
                  pl.BlockSpec((BN, H), lambda i: (i, 0)),
                  pl.BlockSpec((H, 1), lambda i: (0, 0)),
                  pl.BlockSpec((1, 1), lambda i: (0, 0))],
        out_specs=pl.BlockSpec((BN, 1), lambda i: (i, 0)),
        out_shape=jax.ShapeDtypeStruct((N, 1), jnp.float32),
    )(q, y2, dis, b2, g2, be2, x1, wf, bf)


def kernel(x, edge_index, W1, b1, g1, be1, W2, b2, g2, be2, Wf, bf):
    src3d = edge_index[0].astype(jnp.int32).reshape(NW, NCHUNK, C)
    dst3d = edge_index[1].astype(jnp.int32).reshape(NW, NCHUNK, C)
    zeros_rows = jnp.zeros((N, H), jnp.float32)
    zeros_deg = jnp.zeros((N, DEGW), jnp.float32)
    ones_deg = jnp.ones((C, DEGW), jnp.float32)

    b1r = b1.reshape(1, H)
    g1r = g1.reshape(1, H)
    be1r = be1.reshape(1, H)
    b2r = b2.reshape(1, H)
    g2r = g2.reshape(1, H)
    be2r = be2.reshape(1, H)
    bfr = bf.reshape(1, 1)

    sc_degree = _sc_degree_kernel()
    sc_scatter = _sc_scatter_kernel()

    degp = sc_degree(dst3d, ones_deg, zeros_deg)
    xw1 = _tc_matmul(x, W1)
    y1, dis = _tc_scale(degp, xw1)

    p = sc_scatter(y1, src3d, dst3d, zeros_rows)
    x1, y2 = _tc_mid(p, y1, dis, b1r, g1r, be1r, W2)

    q = sc_scatter(y2, src3d, dst3d, zeros_rows)
    return _tc_final(q, y2, dis, b2r, g2r, be2r, x1, Wf, bfr)
